# baseline (speedup 1.0000x reference)
import functools

import jax
import jax.numpy as jnp
from jax import lax
from jax.experimental import pallas as pl
from jax.experimental.pallas import tpu as pltpu
from jax.experimental.pallas import tpu_sc as plsc

N_NODES = 10000
N_EDGES = 320000
IN_CH = 128
HID = 64
OUT_CH = 64

NC, NS = 2, 16
NW = NC * NS
EPW = N_EDGES // NW
HIDH = HID // 2
EPS = N_EDGES // NS
CHUNK = 128
NCHUNK = 160
EPS_PAD = NCHUNK * CHUNK
PADS = EPS_PAD - EPS
NBUF = 4
N_TAB = N_NODES + 8
ROW_BLK = 80
NROWBLK = N_NODES // ROW_BLK
RB = 1000
NBLK = N_NODES // RB

_MESH = plsc.VectorSubcoreMesh(core_axis_name="c", subcore_axis_name="s")


def _worker_ids():
    c = lax.axis_index("c")
    s = lax.axis_index("s")
    return c, s, c * NS + s


def _deg_body(dst_hbm, out_hbm, dstv, degloc, sem):
    c, s, w = _worker_ids()

    zeros = jnp.zeros((16,), jnp.float32)
    for k in range(N_NODES // 16):
        degloc[pl.ds(k * 16, 16)] = zeros

    pltpu.sync_copy(dst_hbm.at[w], dstv)
    ones = jnp.ones((16,), jnp.float32)

    for g in range(EPW // 16):
        idx = dstv[pl.ds(g * 16, 16)]
        plsc.addupdate_scatter(degloc, [idx], ones)

    for k in range(NBLK):
        pltpu.sync_copy(degloc.at[pl.ds(k * RB, RB)], out_hbm.at[k, w])


_sc_deg = pl.kernel(
    _deg_body,
    out_type=jax.ShapeDtypeStruct((NBLK, NW, RB), jnp.float32),
    mesh=_MESH,
    scratch_types=[
        pltpu.VMEM((EPW,), jnp.int32),
        pltpu.VMEM((N_NODES,), jnp.float32),
        pltpu.SemaphoreType.DMA,
    ],
    compiler_params=pltpu.CompilerParams(use_tc_tiling_on_sc=False,
                                         needs_layout_passes=False),
)


def _scatter_body(g_hbm, src_hbm, dst_hbm, zeros_hbm, out_hbm,
                  srcv, dstv, ra0, ra1, ra2, ra3, rb0, rb1, rb2, rb3,
                  zerosv, acc, gsa, gsb, ssa, ssb):
    c, s, w = _worker_ids()
    bufa = (ra0, ra1, ra2, ra3)
    bufb = (rb0, rb1, rb2, rb3)
    pltpu.sync_copy(zeros_hbm, zerosv)

    @pl.loop(s, NROWBLK, step=NS)
    def _zero(k):
        pltpu.sync_copy(zerosv, acc.at[pl.ds(k * ROW_BLK, ROW_BLK)])

    plsc.subcore_barrier()
    pltpu.sync_copy(src_hbm.at[w], srcv)
    pltpu.sync_copy(dst_hbm.at[s], dstv)

    for b in range(NBUF):
        pltpu.async_copy(g_hbm.at[srcv.at[b]], bufa[b], gsa)

    @pl.loop(0, NCHUNK, step=2 * NBUF)
    def _edges(j):
        for b in range(NBUF):
            pltpu.async_copy(g_hbm.at[srcv.at[j + NBUF + b]], bufb[b], gsb)
        for b in range(NBUF):
            pltpu.make_async_copy(g_hbm.at[srcv.at[j + b]], bufa[b], gsa).wait()
        for b in range(NBUF):
            pltpu.async_copy(bufa[b], acc.at[dstv.at[j + b]], ssa, add=True)
        for b in range(NBUF):
            pltpu.make_async_copy(g_hbm.at[srcv.at[j + NBUF + b]], bufb[b], gsb).wait()
        for b in range(NBUF):
            pltpu.async_copy(bufb[b], acc.at[dstv.at[j + NBUF + b]], ssb, add=True)
        for b in range(NBUF):
            pltpu.make_async_copy(bufa[b], acc.at[dstv.at[j + b]], ssa).wait()

        @pl.when(j + 2 * NBUF < NCHUNK)
        def _prefetch_a():
            for b in range(NBUF):
                pltpu.async_copy(g_hbm.at[srcv.at[j + 2 * NBUF + b]], bufa[b], gsa)

        for b in range(NBUF):
            pltpu.make_async_copy(bufb[b], acc.at[dstv.at[j + NBUF + b]], ssb).wait()

    plsc.subcore_barrier()

    @pl.loop(s, NROWBLK, step=NS)
    def _out(k):
        pltpu.sync_copy(acc.at[pl.ds(k * ROW_BLK, ROW_BLK)],
                        out_hbm.at[c, pl.ds(k * ROW_BLK, ROW_BLK)])


_sc_scatter = pl.kernel(
    _scatter_body,
    out_type=jax.ShapeDtypeStruct((NC, N_NODES, HIDH), jnp.float32),
    mesh=_MESH,
    scratch_types=(
        [pltpu.VMEM((NCHUNK, CHUNK), jnp.int32)] * 2
        + [pltpu.VMEM((CHUNK, HIDH), jnp.float32)] * (2 * NBUF)
        + [pltpu.VMEM((ROW_BLK, HIDH), jnp.float32),
           pltpu.VMEM_SHARED((N_NODES, HIDH), jnp.float32)]
        + [pltpu.SemaphoreType.DMA] * 4
    ),
    compiler_params=pltpu.CompilerParams(use_tc_tiling_on_sc=False),
)


GRID = (N_NODES // RB,)
GRID_TAB = (N_NODES // RB + 1,)


def _dinv(dp_ref):
    deg = lax.dot_general(dp_ref[0], jnp.ones((NW, 1), jnp.float32),
                          (((0,), (0,)), ((), ())),
                          preferred_element_type=jnp.float32)
    return lax.rsqrt(1.0 + deg)


def _pad_mask(val):
    last = pl.program_id(0) == NBLK
    return jnp.where(last, 0.0, val)


def _tc0_body(x_ref, w_ref, dp_ref, g_ref):
    g_ref[...] = _pad_mask(_dinv(dp_ref) * jnp.dot(
        x_ref[...], w_ref[...], preferred_element_type=jnp.float32))


def _acc_cat(aa_ref, ab_ref):
    return jnp.concatenate([aa_ref[0], ab_ref[0]], axis=1)


def _tc_mid_body(aa_ref, ab_ref, g_ref, dp_ref, b_ref, w_ref,
                 h_ref, g2_ref):
    dinv = _dinv(dp_ref)
    h = jnp.maximum(
        dinv * (_acc_cat(aa_ref, ab_ref) + g_ref[...]) + b_ref[...], 0.0)
    h_ref[...] = h
    g2_ref[...] = _pad_mask(
        dinv * jnp.dot(h, w_ref[...], preferred_element_type=jnp.float32))


def _tc_fin_body(aa_ref, ab_ref, g_ref, dp_ref, b_ref,
                 x_ref, h1_ref, h2_ref, wx_ref, wh1_ref, wh2_ref, wh3_ref,
                 bm1_ref, wm2_ref, bm2_ref, out_ref):
    dinv = _dinv(dp_ref)
    h3 = jnp.maximum(
        dinv * (_acc_cat(aa_ref, ab_ref) + g_ref[...]) + b_ref[...], 0.0)
    m = jnp.dot(x_ref[...], wx_ref[...], preferred_element_type=jnp.float32)
    m += jnp.dot(h1_ref[...], wh1_ref[...], preferred_element_type=jnp.float32)
    m += jnp.dot(h2_ref[...], wh2_ref[...], preferred_element_type=jnp.float32)
    m += jnp.dot(h3, wh3_ref[...], preferred_element_type=jnp.float32)
    m = jnp.maximum(m + bm1_ref[...], 0.0)
    z = jnp.dot(m, wm2_ref[...], preferred_element_type=jnp.float32) + bm2_ref[...]
    z -= jnp.max(z, axis=1, keepdims=True)
    ez = jnp.exp(z)
    out_ref[...] = ez / jnp.sum(ez, axis=1, keepdims=True)


def _rows(nc):
    return pl.BlockSpec((RB, nc), lambda i: (i, 0))


def _rows_clamped(nc):
    return pl.BlockSpec((RB, nc), lambda i: (jnp.minimum(i, NBLK - 1), 0))


def _full(nr, nc):
    return pl.BlockSpec((nr, nc), lambda i: (0, 0))


def _full_clamped(nr, nc):
    return pl.BlockSpec((nr, nc), lambda i: (0, 0))


def _degp():
    return pl.BlockSpec((1, NW, RB), lambda i: (i, 0, 0))


def _degp_clamped():
    return pl.BlockSpec((1, NW, RB), lambda i: (jnp.minimum(i, NBLK - 1), 0, 0))


def _acc_half(h):
    return pl.BlockSpec((1, RB, HIDH), lambda i, _h=h: (_h, i, 0))


def _acc_half_clamped(h):
    return pl.BlockSpec((1, RB, HIDH),
                        lambda i, _h=h: (_h, jnp.minimum(i, NBLK - 1), 0))


_tc0 = pl.pallas_call(
    _tc0_body,
    grid=GRID_TAB,
    in_specs=[_rows_clamped(IN_CH), _full_clamped(IN_CH, HID),
              _degp_clamped()],
    out_specs=pl.BlockSpec((RB, HID), lambda i: (i, 0)),
    out_shape=jax.ShapeDtypeStruct((N_TAB, HID), jnp.float32),
)

_tc_mid = pl.pallas_call(
    _tc_mid_body,
    grid=GRID_TAB,
    in_specs=[_acc_half_clamped(0), _acc_half_clamped(1), _rows_clamped(HID),
              _degp_clamped(),
              _full_clamped(1, HID), _full_clamped(HID, HID)],
    out_specs=[pl.BlockSpec((RB, HID), lambda i: (jnp.minimum(i, NBLK - 1), 0)),
               pl.BlockSpec((RB, HID), lambda i: (i, 0))],
    out_shape=[jax.ShapeDtypeStruct((N_NODES, HID), jnp.float32),
               jax.ShapeDtypeStruct((N_TAB, HID), jnp.float32)],
)

_tc_fin = pl.pallas_call(
    _tc_fin_body,
    grid=GRID,
    in_specs=[_acc_half(0), _acc_half(1), _rows(HID), _degp(),
              _full(1, HID), _rows(IN_CH), _rows(HID), _rows(HID),
              _full(IN_CH, HID), _full(HID, HID), _full(HID, HID),
              _full(HID, HID), _full(1, HID), _full(HID, OUT_CH),
              _full(1, OUT_CH)],
    out_specs=_rows(OUT_CH),
    out_shape=jax.ShapeDtypeStruct((N_NODES, OUT_CH), jnp.float32),
)


@jax.jit
def kernel(x, edge_index, W1, b1, W2, b2, W3, b3, Wm1, bm1, Wm2, bm2):
    src32 = edge_index[0].astype(jnp.int32).reshape(NS, EPS)
    dst32 = edge_index[1].astype(jnp.int32).reshape(NS, EPS)
    srcp = jnp.concatenate(
        [src32, jnp.full((NS, PADS), N_NODES, jnp.int32)], axis=1)
    srcx = jnp.concatenate(
        [2 * srcp, 2 * srcp + 1], axis=0).reshape(NW, NCHUNK, CHUNK)
    dst = jnp.concatenate(
        [dst32, jnp.zeros((NS, PADS), jnp.int32)],
        axis=1).reshape(NS, NCHUNK, CHUNK)
    dst_d = edge_index[1].astype(jnp.int32).reshape(NW, EPW)

    zeros32 = jnp.zeros((ROW_BLK, HIDH), jnp.float32)

    dp = _sc_deg(dst_d)

    g1 = _tc0(x, W1, dp)
    acc1 = _sc_scatter(g1.reshape(2 * N_TAB, HIDH), srcx, dst, zeros32)
    h1, g2 = _tc_mid(acc1, acc1, g1, dp, b1.reshape(1, HID), W2)
    acc2 = _sc_scatter(g2.reshape(2 * N_TAB, HIDH), srcx, dst, zeros32)
    h2, g3 = _tc_mid(acc2, acc2, g2, dp, b2.reshape(1, HID), W3)
    acc3 = _sc_scatter(g3.reshape(2 * N_TAB, HIDH), srcx, dst, zeros32)

    return _tc_fin(acc3, acc3, g3, dp, b3.reshape(1, HID),
                   x, h1, h2,
                   Wm1[:IN_CH], Wm1[IN_CH:IN_CH + HID],
                   Wm1[IN_CH + HID:IN_CH + 2 * HID], Wm1[IN_CH + 2 * HID:],
                   bm1.reshape(1, HID), Wm2, bm2.reshape(1, OUT_CH))


# speedup vs baseline: 2.1433x; 2.1433x over previous
import functools

import jax
import jax.numpy as jnp
from jax import lax
from jax.experimental import pallas as pl
from jax.experimental.pallas import tpu as pltpu
from jax.experimental.pallas import tpu_sc as plsc

N_NODES = 10000
N_EDGES = 320000
IN_CH = 128
HID = 64
OUT_CH = 64

NC, NS = 2, 16
NW = NC * NS
EPW = N_EDGES // NW
CHUNK = 125
NCHUNK = 80
NBUF = 4
ROW_BLK = 80
NROWBLK = N_NODES // ROW_BLK
RB = 1000
NBLK = N_NODES // RB

_MESH = plsc.VectorSubcoreMesh(core_axis_name="c", subcore_axis_name="s")


def _worker_ids():
    c = lax.axis_index("c")
    s = lax.axis_index("s")
    return c, s, c * NS + s


def _deg_body(dst_hbm, out_hbm, dstv, degloc, sem):
    c, s, w = _worker_ids()

    zeros = jnp.zeros((16,), jnp.float32)
    for k in range(N_NODES // 16):
        degloc[pl.ds(k * 16, 16)] = zeros

    pltpu.sync_copy(dst_hbm.at[w], dstv)
    ones = jnp.ones((16,), jnp.float32)

    for g in range(EPW // 16):
        idx = dstv[pl.ds(g * 16, 16)]
        plsc.addupdate_scatter(degloc, [idx], ones)

    for k in range(NBLK):
        pltpu.sync_copy(degloc.at[pl.ds(k * RB, RB)], out_hbm.at[k, w])


_sc_deg = pl.kernel(
    _deg_body,
    out_type=jax.ShapeDtypeStruct((NBLK, NW, RB), jnp.float32),
    mesh=_MESH,
    scratch_types=[
        pltpu.VMEM((EPW,), jnp.int32),
        pltpu.VMEM((N_NODES,), jnp.float32),
        pltpu.SemaphoreType.DMA,
    ],
    compiler_params=pltpu.CompilerParams(use_tc_tiling_on_sc=False,
                                         needs_layout_passes=False),
)


def _scatter_body(g_hbm, src_hbm, dst_hbm, zeros_hbm, out_hbm,
                  srcv, dstv, ra0, ra1, ra2, ra3, rb0, rb1, rb2, rb3,
                  zerosv, acc, gsa, gsb, ssa, ssb):
    c, s, w = _worker_ids()
    bufa = (ra0, ra1, ra2, ra3)
    bufb = (rb0, rb1, rb2, rb3)
    pltpu.sync_copy(zeros_hbm, zerosv)

    @pl.loop(s, NROWBLK, step=NS)
    def _zero(k):
        pltpu.sync_copy(zerosv, acc.at[pl.ds(k * ROW_BLK, ROW_BLK)])

    plsc.subcore_barrier()
    pltpu.sync_copy(src_hbm.at[w], srcv)
    pltpu.sync_copy(dst_hbm.at[w], dstv)

    for b in range(NBUF):
        pltpu.async_copy(g_hbm.at[srcv.at[b]], bufa[b], gsa)

    @pl.loop(0, NCHUNK, step=2 * NBUF)
    def _edges(j):
        for b in range(NBUF):
            pltpu.async_copy(g_hbm.at[srcv.at[j + NBUF + b]], bufb[b], gsb)
        for b in range(NBUF):
            pltpu.make_async_copy(g_hbm.at[srcv.at[j + b]], bufa[b], gsa).wait()
        for b in range(NBUF):
            pltpu.async_copy(bufa[b], acc.at[dstv.at[j + b]], ssa, add=True)
        for b in range(NBUF):
            pltpu.make_async_copy(g_hbm.at[srcv.at[j + NBUF + b]], bufb[b], gsb).wait()
        for b in range(NBUF):
            pltpu.async_copy(bufb[b], acc.at[dstv.at[j + NBUF + b]], ssb, add=True)
        for b in range(NBUF):
            pltpu.make_async_copy(bufa[b], acc.at[dstv.at[j + b]], ssa).wait()

        @pl.when(j + 2 * NBUF < NCHUNK)
        def _prefetch_a():
            for b in range(NBUF):
                pltpu.async_copy(g_hbm.at[srcv.at[j + 2 * NBUF + b]], bufa[b], gsa)

        for b in range(NBUF):
            pltpu.make_async_copy(bufb[b], acc.at[dstv.at[j + NBUF + b]], ssb).wait()

    plsc.subcore_barrier()

    @pl.loop(s, NROWBLK, step=NS)
    def _out(k):
        pltpu.sync_copy(acc.at[pl.ds(k * ROW_BLK, ROW_BLK)],
                        out_hbm.at[c, pl.ds(k * ROW_BLK, ROW_BLK)])


_sc_scatter = pl.kernel(
    _scatter_body,
    out_type=jax.ShapeDtypeStruct((NC, N_NODES, HID), jnp.float32),
    mesh=_MESH,
    scratch_types=(
        [pltpu.VMEM((NCHUNK, CHUNK), jnp.int32)] * 2
        + [pltpu.VMEM((CHUNK, HID), jnp.float32)] * (2 * NBUF)
        + [pltpu.VMEM((ROW_BLK, HID), jnp.float32),
           pltpu.VMEM_SHARED((N_NODES, HID), jnp.float32)]
        + [pltpu.SemaphoreType.DMA] * 4
    ),
    compiler_params=pltpu.CompilerParams(use_tc_tiling_on_sc=False),
)


GRID = (N_NODES // RB,)


def _dinv(dp_ref):
    deg = lax.dot_general(dp_ref[0], jnp.ones((NW, 1), jnp.float32),
                          (((0,), (0,)), ((), ())),
                          preferred_element_type=jnp.float32)
    return lax.rsqrt(1.0 + deg)


def _tc0_body(x_ref, w_ref, dp_ref, g_ref):
    g_ref[...] = _dinv(dp_ref) * jnp.dot(
        x_ref[...], w_ref[...], preferred_element_type=jnp.float32)


def _acc_sum(aa_ref, ab_ref):
    return aa_ref[0] + ab_ref[0]


def _tc_mid_body(aa_ref, ab_ref, g_ref, dp_ref, b_ref, w_ref,
                 h_ref, g2_ref):
    dinv = _dinv(dp_ref)
    h = jnp.maximum(
        dinv * (_acc_sum(aa_ref, ab_ref) + g_ref[...]) + b_ref[...], 0.0)
    h_ref[...] = h
    g2_ref[...] = dinv * jnp.dot(h, w_ref[...],
                                 preferred_element_type=jnp.float32)


def _tc_fin_body(aa_ref, ab_ref, g_ref, dp_ref, b_ref,
                 x_ref, h1_ref, h2_ref, wx_ref, wh1_ref, wh2_ref, wh3_ref,
                 bm1_ref, wm2_ref, bm2_ref, out_ref):
    dinv = _dinv(dp_ref)
    h3 = jnp.maximum(
        dinv * (_acc_sum(aa_ref, ab_ref) + g_ref[...]) + b_ref[...], 0.0)
    m = jnp.dot(x_ref[...], wx_ref[...], preferred_element_type=jnp.float32)
    m += jnp.dot(h1_ref[...], wh1_ref[...], preferred_element_type=jnp.float32)
    m += jnp.dot(h2_ref[...], wh2_ref[...], preferred_element_type=jnp.float32)
    m += jnp.dot(h3, wh3_ref[...], preferred_element_type=jnp.float32)
    m = jnp.maximum(m + bm1_ref[...], 0.0)
    z = jnp.dot(m, wm2_ref[...], preferred_element_type=jnp.float32) + bm2_ref[...]
    z -= jnp.max(z, axis=1, keepdims=True)
    ez = jnp.exp(z)
    out_ref[...] = ez / jnp.sum(ez, axis=1, keepdims=True)


def _rows(nc):
    return pl.BlockSpec((RB, nc), lambda i: (i, 0))


def _full(nr, nc):
    return pl.BlockSpec((nr, nc), lambda i: (0, 0))


def _degp():
    return pl.BlockSpec((1, NW, RB), lambda i: (i, 0, 0))


def _acc_half(h):
    return pl.BlockSpec((1, RB, HID), lambda i, _h=h: (_h, i, 0))


_tc0 = pl.pallas_call(
    _tc0_body,
    grid=GRID,
    in_specs=[_rows(IN_CH), _full(IN_CH, HID), _degp()],
    out_specs=_rows(HID),
    out_shape=jax.ShapeDtypeStruct((N_NODES, HID), jnp.float32),
)

_tc_mid = pl.pallas_call(
    _tc_mid_body,
    grid=GRID,
    in_specs=[_acc_half(0), _acc_half(1), _rows(HID), _degp(),
              _full(1, HID), _full(HID, HID)],
    out_specs=[_rows(HID), _rows(HID)],
    out_shape=[jax.ShapeDtypeStruct((N_NODES, HID), jnp.float32),
               jax.ShapeDtypeStruct((N_NODES, HID), jnp.float32)],
)

_tc_fin = pl.pallas_call(
    _tc_fin_body,
    grid=GRID,
    in_specs=[_acc_half(0), _acc_half(1), _rows(HID), _degp(),
              _full(1, HID), _rows(IN_CH), _rows(HID), _rows(HID),
              _full(IN_CH, HID), _full(HID, HID), _full(HID, HID),
              _full(HID, HID), _full(1, HID), _full(HID, OUT_CH),
              _full(1, OUT_CH)],
    out_specs=_rows(OUT_CH),
    out_shape=jax.ShapeDtypeStruct((N_NODES, OUT_CH), jnp.float32),
)


@jax.jit
def kernel(x, edge_index, W1, b1, W2, b2, W3, b3, Wm1, bm1, Wm2, bm2):
    src = edge_index[0].astype(jnp.int32).reshape(NW, NCHUNK, CHUNK)
    dst = edge_index[1].astype(jnp.int32).reshape(NW, NCHUNK, CHUNK)
    dst_d = edge_index[1].astype(jnp.int32).reshape(NW, EPW)

    zeros64 = jnp.zeros((ROW_BLK, HID), jnp.float32)

    dp = _sc_deg(dst_d)

    g1 = _tc0(x, W1, dp)
    acc1 = _sc_scatter(g1, src, dst, zeros64)
    h1, g2 = _tc_mid(acc1, acc1, g1, dp, b1.reshape(1, HID), W2)
    acc2 = _sc_scatter(g2, src, dst, zeros64)
    h2, g3 = _tc_mid(acc2, acc2, g2, dp, b2.reshape(1, HID), W3)
    acc3 = _sc_scatter(g3, src, dst, zeros64)

    return _tc_fin(acc3, acc3, g3, dp, b3.reshape(1, HID),
                   x, h1, h2,
                   Wm1[:IN_CH], Wm1[IN_CH:IN_CH + HID],
                   Wm1[IN_CH + HID:IN_CH + 2 * HID], Wm1[IN_CH + 2 * HID:],
                   bm1.reshape(1, HID), Wm2, bm2.reshape(1, OUT_CH))


# speedup vs baseline: 2.4434x; 1.1400x over previous
import functools

import jax
import jax.numpy as jnp
from jax import lax
from jax.experimental import pallas as pl
from jax.experimental.pallas import tpu as pltpu
from jax.experimental.pallas import tpu_sc as plsc

N_NODES = 10000
N_EDGES = 320000
IN_CH = 128
HID = 64
OUT_CH = 64

NC, NS = 2, 16
NW = NC * NS
EPW = N_EDGES // NW
CHUNK = 125
NCHUNK = 80
NBUF = 4
ROW_BLK = 80
NROWBLK = N_NODES // ROW_BLK
RB = 2000
NBLK = N_NODES // RB
RBH = RB // 2
NH = N_NODES // 2
NHP = NH + 8

_MESH = plsc.VectorSubcoreMesh(core_axis_name="c", subcore_axis_name="s")


def _worker_ids():
    c = lax.axis_index("c")
    s = lax.axis_index("s")
    return c, s, c * NS + s


def _deg_body(dst_hbm, out_hbm, dstv, dege, dego, sem):
    c, s, w = _worker_ids()

    zeros = jnp.zeros((16,), jnp.float32)
    for k in range(NHP // 16):
        dege[pl.ds(k * 16, 16)] = zeros
        dego[pl.ds(k * 16, 16)] = zeros

    pltpu.sync_copy(dst_hbm.at[w], dstv)
    ones = jnp.ones((16,), jnp.float32)

    for g in range(EPW // 16):
        idx = dstv[pl.ds(g * 16, 16)]
        row = jax.lax.shift_right_logical(idx, 1)
        odd = jax.lax.eq(jax.lax.bitwise_and(idx, 1), 1)
        plsc.addupdate_scatter(dege, [row], ones, mask=jnp.logical_not(odd))
        plsc.addupdate_scatter(dego, [row], ones, mask=odd)

    for k in range(NBLK):
        pltpu.sync_copy(dege.at[pl.ds(k * RBH, RBH)], out_hbm.at[k, w, 0])
        pltpu.sync_copy(dego.at[pl.ds(k * RBH, RBH)], out_hbm.at[k, w, 1])


_sc_deg = pl.kernel(
    _deg_body,
    out_type=jax.ShapeDtypeStruct((NBLK, NW, 2, RBH), jnp.float32),
    mesh=_MESH,
    scratch_types=[
        pltpu.VMEM((EPW,), jnp.int32),
        pltpu.VMEM((NHP,), jnp.float32),
        pltpu.VMEM((NHP,), jnp.float32),
        pltpu.SemaphoreType.DMA,
    ],
    compiler_params=pltpu.CompilerParams(use_tc_tiling_on_sc=False,
                                         needs_layout_passes=False),
)


def _scatter_body(g_hbm, src_hbm, dst_hbm, zeros_hbm, out_hbm,
                  srcv, dstv, ra0, ra1, ra2, ra3, rb0, rb1, rb2, rb3,
                  zerosv, acc, gsa, gsb, ssa, ssb):
    c, s, w = _worker_ids()
    bufa = (ra0, ra1, ra2, ra3)
    bufb = (rb0, rb1, rb2, rb3)
    pltpu.sync_copy(zeros_hbm, zerosv)

    @pl.loop(s, NROWBLK, step=NS)
    def _zero(k):
        pltpu.sync_copy(zerosv, acc.at[pl.ds(k * ROW_BLK, ROW_BLK)])

    plsc.subcore_barrier()
    pltpu.sync_copy(src_hbm.at[w], srcv)
    pltpu.sync_copy(dst_hbm.at[w], dstv)

    for b in range(NBUF):
        pltpu.async_copy(g_hbm.at[srcv.at[b]], bufa[b], gsa)

    @pl.loop(0, NCHUNK, step=2 * NBUF)
    def _edges(j):
        for b in range(NBUF):
            pltpu.async_copy(g_hbm.at[srcv.at[j + NBUF + b]], bufb[b], gsb)
        for b in range(NBUF):
            pltpu.make_async_copy(g_hbm.at[srcv.at[j + b]], bufa[b], gsa).wait()
        for b in range(NBUF):
            pltpu.async_copy(bufa[b], acc.at[dstv.at[j + b]], ssa, add=True)
        for b in range(NBUF):
            pltpu.make_async_copy(g_hbm.at[srcv.at[j + NBUF + b]], bufb[b], gsb).wait()
        for b in range(NBUF):
            pltpu.async_copy(bufb[b], acc.at[dstv.at[j + NBUF + b]], ssb, add=True)
        for b in range(NBUF):
            pltpu.make_async_copy(bufa[b], acc.at[dstv.at[j + b]], ssa).wait()

        @pl.when(j + 2 * NBUF < NCHUNK)
        def _prefetch_a():
            for b in range(NBUF):
                pltpu.async_copy(g_hbm.at[srcv.at[j + 2 * NBUF + b]], bufa[b], gsa)

        for b in range(NBUF):
            pltpu.make_async_copy(bufb[b], acc.at[dstv.at[j + NBUF + b]], ssb).wait()

    plsc.subcore_barrier()

    @pl.loop(s, NROWBLK, step=NS)
    def _out(k):
        pltpu.sync_copy(acc.at[pl.ds(k * ROW_BLK, ROW_BLK)],
                        out_hbm.at[c, pl.ds(k * ROW_BLK, ROW_BLK)])


_sc_scatter = pl.kernel(
    _scatter_body,
    out_type=jax.ShapeDtypeStruct((NC, N_NODES, HID), jnp.float32),
    mesh=_MESH,
    scratch_types=(
        [pltpu.VMEM((NCHUNK, CHUNK), jnp.int32)] * 2
        + [pltpu.VMEM((CHUNK, HID), jnp.float32)] * (2 * NBUF)
        + [pltpu.VMEM((ROW_BLK, HID), jnp.float32),
           pltpu.VMEM_SHARED((N_NODES, HID), jnp.float32)]
        + [pltpu.SemaphoreType.DMA] * 4
    ),
    compiler_params=pltpu.CompilerParams(use_tc_tiling_on_sc=False),
)


GRID = (N_NODES // RB,)


def _dinv_pair(dp_ref):
    ones = jnp.ones((NW, 1), jnp.float32)
    dn = (((0,), (0,)), ((), ()))
    de = lax.dot_general(dp_ref[0, :, 0], ones, dn,
                         preferred_element_type=jnp.float32)
    do = lax.dot_general(dp_ref[0, :, 1], ones, dn,
                         preferred_element_type=jnp.float32)
    return lax.rsqrt(1.0 + de), lax.rsqrt(1.0 + do)


def _halves(ref):
    v = ref[...]
    n = v.shape[1] // 2
    return v[:, :n], v[:, n:]


def _tc0_body(x_ref, w_ref, dp_ref, g_ref):
    dve, dvo = _dinv_pair(dp_ref)
    xe, xo = _halves(x_ref)
    ge = dve * jnp.dot(xe, w_ref[...], preferred_element_type=jnp.float32)
    go = dvo * jnp.dot(xo, w_ref[...], preferred_element_type=jnp.float32)
    g_ref[...] = jnp.concatenate([ge, go], axis=1)


def _acc_halves(aa_ref, ab_ref):
    v = aa_ref[0] + ab_ref[0]
    return v[:, :HID], v[:, HID:]


def _tc_mid_body(aa_ref, ab_ref, g_ref, dp_ref, b_ref, w_ref,
                 h_ref, g2_ref):
    dve, dvo = _dinv_pair(dp_ref)
    ae, ao = _acc_halves(aa_ref, ab_ref)
    ge, go = _halves(g_ref)
    he = jnp.maximum(dve * (ae + ge) + b_ref[...], 0.0)
    ho = jnp.maximum(dvo * (ao + go) + b_ref[...], 0.0)
    h_ref[...] = jnp.concatenate([he, ho], axis=1)
    g2e = dve * jnp.dot(he, w_ref[...], preferred_element_type=jnp.float32)
    g2o = dvo * jnp.dot(ho, w_ref[...], preferred_element_type=jnp.float32)
    g2_ref[...] = jnp.concatenate([g2e, g2o], axis=1)


def _softmax(z):
    z -= jnp.max(z, axis=1, keepdims=True)
    ez = jnp.exp(z)
    return ez / jnp.sum(ez, axis=1, keepdims=True)


def _tc_fin_body(aa_ref, ab_ref, g_ref, dp_ref, b_ref,
                 x_ref, h1_ref, h2_ref, wx_ref, wh1_ref, wh2_ref, wh3_ref,
                 bm1_ref, wm2_ref, bm2_ref, out_ref):
    dve, dvo = _dinv_pair(dp_ref)
    ae, ao = _acc_halves(aa_ref, ab_ref)
    ge, go = _halves(g_ref)
    h3e = jnp.maximum(dve * (ae + ge) + b_ref[...], 0.0)
    h3o = jnp.maximum(dvo * (ao + go) + b_ref[...], 0.0)
    xe, xo = _halves(x_ref)
    h1e, h1o = _halves(h1_ref)
    h2e, h2o = _halves(h2_ref)
    f32 = jnp.float32
    me = (jnp.dot(xe, wx_ref[...], preferred_element_type=f32)
          + jnp.dot(h1e, wh1_ref[...], preferred_element_type=f32)
          + jnp.dot(h2e, wh2_ref[...], preferred_element_type=f32)
          + jnp.dot(h3e, wh3_ref[...], preferred_element_type=f32))
    mo = (jnp.dot(xo, wx_ref[...], preferred_element_type=f32)
          + jnp.dot(h1o, wh1_ref[...], preferred_element_type=f32)
          + jnp.dot(h2o, wh2_ref[...], preferred_element_type=f32)
          + jnp.dot(h3o, wh3_ref[...], preferred_element_type=f32))
    me = jnp.maximum(me + bm1_ref[...], 0.0)
    mo = jnp.maximum(mo + bm1_ref[...], 0.0)
    ze = jnp.dot(me, wm2_ref[...], preferred_element_type=f32) + bm2_ref[...]
    zo = jnp.dot(mo, wm2_ref[...], preferred_element_type=f32) + bm2_ref[...]
    out_ref[...] = jnp.concatenate([_softmax(ze), _softmax(zo)], axis=1)


def _rows(nc):
    return pl.BlockSpec((RB, nc), lambda i: (i, 0))


def _full(nr, nc):
    return pl.BlockSpec((nr, nc), lambda i: (0, 0))


def _degp():
    return pl.BlockSpec((1, NW, 2, RBH), lambda i: (i, 0, 0, 0))


def _acc_half(h):
    return pl.BlockSpec((1, RBH, 2 * HID), lambda i, _h=h: (_h, i, 0))


def _packed(nc):
    return pl.BlockSpec((RBH, 2 * nc), lambda i: (i, 0))


_tc0 = pl.pallas_call(
    _tc0_body,
    grid=GRID,
    in_specs=[_packed(IN_CH), _full(IN_CH, HID), _degp()],
    out_specs=_packed(HID),
    out_shape=jax.ShapeDtypeStruct((NH, 2 * HID), jnp.float32),
)

_tc_mid = pl.pallas_call(
    _tc_mid_body,
    grid=GRID,
    in_specs=[_acc_half(0), _acc_half(1), _packed(HID), _degp(),
              _full(1, HID), _full(HID, HID)],
    out_specs=[_packed(HID), _packed(HID)],
    out_shape=[jax.ShapeDtypeStruct((NH, 2 * HID), jnp.float32),
               jax.ShapeDtypeStruct((NH, 2 * HID), jnp.float32)],
)

_tc_fin = pl.pallas_call(
    _tc_fin_body,
    grid=GRID,
    in_specs=[_acc_half(0), _acc_half(1), _packed(HID), _degp(),
              _full(1, HID), _packed(IN_CH), _packed(HID), _packed(HID),
              _full(IN_CH, HID), _full(HID, HID), _full(HID, HID),
              _full(HID, HID), _full(1, HID), _full(HID, OUT_CH),
              _full(1, OUT_CH)],
    out_specs=_packed(OUT_CH),
    out_shape=jax.ShapeDtypeStruct((NH, 2 * OUT_CH), jnp.float32),
)


@jax.jit
def kernel(x, edge_index, W1, b1, W2, b2, W3, b3, Wm1, bm1, Wm2, bm2):
    src = edge_index[0].astype(jnp.int32).reshape(NW, NCHUNK, CHUNK)
    dst = edge_index[1].astype(jnp.int32).reshape(NW, NCHUNK, CHUNK)
    dst_d = edge_index[1].astype(jnp.int32).reshape(NW, EPW)

    zeros64 = jnp.zeros((ROW_BLK, HID), jnp.float32)

    dp = _sc_deg(dst_d)

    def unview(g):
        return g.reshape(N_NODES, HID)

    def view(a):
        return a.reshape(NC, NH, 2 * HID)

    xp = x.reshape(NH, 2 * IN_CH)

    g1 = _tc0(xp, W1, dp)
    acc1 = view(_sc_scatter(unview(g1), src, dst, zeros64))
    h1, g2 = _tc_mid(acc1, acc1, g1, dp, b1.reshape(1, HID), W2)
    acc2 = view(_sc_scatter(unview(g2), src, dst, zeros64))
    h2, g3 = _tc_mid(acc2, acc2, g2, dp, b2.reshape(1, HID), W3)
    acc3 = view(_sc_scatter(unview(g3), src, dst, zeros64))

    out = _tc_fin(acc3, acc3, g3, dp, b3.reshape(1, HID),
                  xp, h1, h2,
                  Wm1[:IN_CH], Wm1[IN_CH:IN_CH + HID],
                  Wm1[IN_CH + HID:IN_CH + 2 * HID], Wm1[IN_CH + 2 * HID:],
                  bm1.reshape(1, HID), Wm2, bm2.reshape(1, OUT_CH))
    return out.reshape(N_NODES, OUT_CH)


# speedup vs baseline: 2.7745x; 1.1355x over previous
import functools

import jax
import jax.numpy as jnp
from jax import lax
from jax.experimental import pallas as pl
from jax.experimental.pallas import tpu as pltpu
from jax.experimental.pallas import tpu_sc as plsc

N_NODES = 10000
N_EDGES = 320000
IN_CH = 128
HID = 64
OUT_CH = 64

NC, NS = 2, 16
NW = NC * NS
EPW = N_EDGES // NW
CHUNK = 125
NCHUNK = 80
NBUF = 4
ROW_BLK = 80
NROWBLK = N_NODES // ROW_BLK
RB = 2000
NBLK = N_NODES // RB
RBH = RB // 2
NH = N_NODES // 2
NHP = NH + 8

_MESH = plsc.VectorSubcoreMesh(core_axis_name="c", subcore_axis_name="s")


def _worker_ids():
    c = lax.axis_index("c")
    s = lax.axis_index("s")
    return c, s, c * NS + s


def _deg_body(dst_hbm, out_hbm, dstv, dege, dego, sem):
    c, s, w = _worker_ids()

    zeros = jnp.zeros((16,), jnp.float32)
    for k in range(NHP // 16):
        dege[pl.ds(k * 16, 16)] = zeros
        dego[pl.ds(k * 16, 16)] = zeros

    pltpu.sync_copy(dst_hbm.at[w], dstv)
    ones = jnp.ones((16,), jnp.float32)

    for g in range(EPW // 16):
        idx = dstv[pl.ds(g * 16, 16)]
        row = jax.lax.shift_right_logical(idx, 1)
        odd = jax.lax.eq(jax.lax.bitwise_and(idx, 1), 1)
        plsc.addupdate_scatter(dege, [row], ones, mask=jnp.logical_not(odd))
        plsc.addupdate_scatter(dego, [row], ones, mask=odd)

    for k in range(NBLK):
        pltpu.sync_copy(dege.at[pl.ds(k * RBH, RBH)], out_hbm.at[k, w, 0])
        pltpu.sync_copy(dego.at[pl.ds(k * RBH, RBH)], out_hbm.at[k, w, 1])


_sc_deg = pl.kernel(
    _deg_body,
    out_type=jax.ShapeDtypeStruct((NBLK, NW, 2, RBH), jnp.float32),
    mesh=_MESH,
    scratch_types=[
        pltpu.VMEM((EPW,), jnp.int32),
        pltpu.VMEM((NHP,), jnp.float32),
        pltpu.VMEM((NHP,), jnp.float32),
        pltpu.SemaphoreType.DMA,
    ],
    compiler_params=pltpu.CompilerParams(use_tc_tiling_on_sc=False,
                                         needs_layout_passes=False),
)


def _scatter_body(g_hbm, src_hbm, dst_hbm, zeros_hbm, out_hbm,
                  srcv, dstv, ra0, ra1, ra2, ra3, rb0, rb1, rb2, rb3,
                  zerosv, acc, gsa, gsb, ssa, ssb):
    c, s, w = _worker_ids()
    bufa = (ra0, ra1, ra2, ra3)
    bufb = (rb0, rb1, rb2, rb3)
    cps = pltpu.async_copy(src_hbm.at[w], srcv, gsa)
    cpd = pltpu.async_copy(dst_hbm.at[w], dstv, gsb)
    pltpu.sync_copy(zeros_hbm, zerosv)
    cps.wait()
    cpd.wait()

    for b in range(NBUF):
        pltpu.async_copy(g_hbm.at[srcv.at[b]], bufa[b], gsa)

    @pl.loop(s, NROWBLK, step=NS)
    def _zero(k):
        pltpu.sync_copy(zerosv, acc.at[pl.ds(k * ROW_BLK, ROW_BLK)])

    plsc.subcore_barrier()

    @pl.loop(0, NCHUNK, step=2 * NBUF)
    def _edges(j):
        for b in range(NBUF):
            pltpu.async_copy(g_hbm.at[srcv.at[j + NBUF + b]], bufb[b], gsb)
        for b in range(NBUF):
            pltpu.make_async_copy(g_hbm.at[srcv.at[j + b]], bufa[b], gsa).wait()
        for b in range(NBUF):
            pltpu.async_copy(bufa[b], acc.at[dstv.at[j + b]], ssa, add=True)
        for b in range(NBUF):
            pltpu.make_async_copy(g_hbm.at[srcv.at[j + NBUF + b]], bufb[b], gsb).wait()
        for b in range(NBUF):
            pltpu.async_copy(bufb[b], acc.at[dstv.at[j + NBUF + b]], ssb, add=True)
        for b in range(NBUF):
            pltpu.make_async_copy(bufa[b], acc.at[dstv.at[j + b]], ssa).wait()

        @pl.when(j + 2 * NBUF < NCHUNK)
        def _prefetch_a():
            for b in range(NBUF):
                pltpu.async_copy(g_hbm.at[srcv.at[j + 2 * NBUF + b]], bufa[b], gsa)

        for b in range(NBUF):
            pltpu.make_async_copy(bufb[b], acc.at[dstv.at[j + NBUF + b]], ssb).wait()

    plsc.subcore_barrier()

    @pl.loop(s, NROWBLK, step=NS)
    def _out(k):
        pltpu.sync_copy(acc.at[pl.ds(k * ROW_BLK, ROW_BLK)],
                        out_hbm.at[c, pl.ds(k * ROW_BLK, ROW_BLK)])


_sc_scatter = pl.kernel(
    _scatter_body,
    out_type=jax.ShapeDtypeStruct((NC, N_NODES, HID), jnp.float32),
    mesh=_MESH,
    scratch_types=(
        [pltpu.VMEM((NCHUNK, CHUNK), jnp.int32)] * 2
        + [pltpu.VMEM((CHUNK, HID), jnp.float32)] * (2 * NBUF)
        + [pltpu.VMEM((ROW_BLK, HID), jnp.float32),
           pltpu.VMEM_SHARED((N_NODES, HID), jnp.float32)]
        + [pltpu.SemaphoreType.DMA] * 4
    ),
    compiler_params=pltpu.CompilerParams(use_tc_tiling_on_sc=False),
)


GRID = (N_NODES // RB,)


def _dinv_pair(dp_ref):
    ones = jnp.ones((NW, 1), jnp.float32)
    dn = (((0,), (0,)), ((), ()))
    de = lax.dot_general(dp_ref[0, :, 0], ones, dn,
                         preferred_element_type=jnp.float32)
    do = lax.dot_general(dp_ref[0, :, 1], ones, dn,
                         preferred_element_type=jnp.float32)
    return lax.rsqrt(1.0 + de), lax.rsqrt(1.0 + do)


def _halves(ref):
    v = ref[...]
    n = v.shape[1] // 2
    return v[:, :n], v[:, n:]


def _tc0_body(x_ref, w_ref, dp_ref, g_ref):
    dve, dvo = _dinv_pair(dp_ref)
    xe, xo = _halves(x_ref)
    ge = dve * jnp.dot(xe, w_ref[...], preferred_element_type=jnp.float32)
    go = dvo * jnp.dot(xo, w_ref[...], preferred_element_type=jnp.float32)
    g_ref[...] = jnp.concatenate([ge, go], axis=1)


def _acc_halves(aa_ref, ab_ref):
    v = aa_ref[0] + ab_ref[0]
    return v[:, :HID], v[:, HID:]


def _tc_mid_body(aa_ref, ab_ref, g_ref, dp_ref, b_ref, w_ref,
                 h_ref, g2_ref):
    dve, dvo = _dinv_pair(dp_ref)
    ae, ao = _acc_halves(aa_ref, ab_ref)
    ge, go = _halves(g_ref)
    he = jnp.maximum(dve * (ae + ge) + b_ref[...], 0.0)
    ho = jnp.maximum(dvo * (ao + go) + b_ref[...], 0.0)
    h_ref[...] = jnp.concatenate([he, ho], axis=1)
    g2e = dve * jnp.dot(he, w_ref[...], preferred_element_type=jnp.float32)
    g2o = dvo * jnp.dot(ho, w_ref[...], preferred_element_type=jnp.float32)
    g2_ref[...] = jnp.concatenate([g2e, g2o], axis=1)


def _softmax(z):
    z -= jnp.max(z, axis=1, keepdims=True)
    ez = jnp.exp(z)
    return ez / jnp.sum(ez, axis=1, keepdims=True)


def _tc_fin_body(aa_ref, ab_ref, g_ref, dp_ref, b_ref,
                 x_ref, h1_ref, h2_ref, wx_ref, wh1_ref, wh2_ref, wh3_ref,
                 bm1_ref, wm2_ref, bm2_ref, out_ref):
    dve, dvo = _dinv_pair(dp_ref)
    ae, ao = _acc_halves(aa_ref, ab_ref)
    ge, go = _halves(g_ref)
    h3e = jnp.maximum(dve * (ae + ge) + b_ref[...], 0.0)
    h3o = jnp.maximum(dvo * (ao + go) + b_ref[...], 0.0)
    xe, xo = _halves(x_ref)
    h1e, h1o = _halves(h1_ref)
    h2e, h2o = _halves(h2_ref)
    f32 = jnp.float32
    me = (jnp.dot(xe, wx_ref[...], preferred_element_type=f32)
          + jnp.dot(h1e, wh1_ref[...], preferred_element_type=f32)
          + jnp.dot(h2e, wh2_ref[...], preferred_element_type=f32)
          + jnp.dot(h3e, wh3_ref[...], preferred_element_type=f32))
    mo = (jnp.dot(xo, wx_ref[...], preferred_element_type=f32)
          + jnp.dot(h1o, wh1_ref[...], preferred_element_type=f32)
          + jnp.dot(h2o, wh2_ref[...], preferred_element_type=f32)
          + jnp.dot(h3o, wh3_ref[...], preferred_element_type=f32))
    me = jnp.maximum(me + bm1_ref[...], 0.0)
    mo = jnp.maximum(mo + bm1_ref[...], 0.0)
    ze = jnp.dot(me, wm2_ref[...], preferred_element_type=f32) + bm2_ref[...]
    zo = jnp.dot(mo, wm2_ref[...], preferred_element_type=f32) + bm2_ref[...]
    out_ref[...] = jnp.concatenate([_softmax(ze), _softmax(zo)], axis=1)


def _rows(nc):
    return pl.BlockSpec((RB, nc), lambda i: (i, 0))


def _full(nr, nc):
    return pl.BlockSpec((nr, nc), lambda i: (0, 0))


def _degp():
    return pl.BlockSpec((1, NW, 2, RBH), lambda i: (i, 0, 0, 0))


def _acc_half(h):
    return pl.BlockSpec((1, RBH, 2 * HID), lambda i, _h=h: (_h, i, 0))


def _packed(nc):
    return pl.BlockSpec((RBH, 2 * nc), lambda i: (i, 0))


_tc0 = pl.pallas_call(
    _tc0_body,
    grid=GRID,
    in_specs=[_packed(IN_CH), _full(IN_CH, HID), _degp()],
    out_specs=_packed(HID),
    out_shape=jax.ShapeDtypeStruct((NH, 2 * HID), jnp.float32),
)

_tc_mid = pl.pallas_call(
    _tc_mid_body,
    grid=GRID,
    in_specs=[_acc_half(0), _acc_half(1), _packed(HID), _degp(),
              _full(1, HID), _full(HID, HID)],
    out_specs=[_packed(HID), _packed(HID)],
    out_shape=[jax.ShapeDtypeStruct((NH, 2 * HID), jnp.float32),
               jax.ShapeDtypeStruct((NH, 2 * HID), jnp.float32)],
)

_tc_fin = pl.pallas_call(
    _tc_fin_body,
    grid=GRID,
    in_specs=[_acc_half(0), _acc_half(1), _packed(HID), _degp(),
              _full(1, HID), _packed(IN_CH), _packed(HID), _packed(HID),
              _full(IN_CH, HID), _full(HID, HID), _full(HID, HID),
              _full(HID, HID), _full(1, HID), _full(HID, OUT_CH),
              _full(1, OUT_CH)],
    out_specs=_packed(OUT_CH),
    out_shape=jax.ShapeDtypeStruct((NH, 2 * OUT_CH), jnp.float32),
)


@jax.jit
def kernel(x, edge_index, W1, b1, W2, b2, W3, b3, Wm1, bm1, Wm2, bm2):
    src = edge_index[0].astype(jnp.int32).reshape(NW, NCHUNK, CHUNK)
    dst = edge_index[1].astype(jnp.int32).reshape(NW, NCHUNK, CHUNK)
    dst_d = edge_index[1].astype(jnp.int32).reshape(NW, EPW)

    zeros64 = jnp.zeros((ROW_BLK, HID), jnp.float32)

    dp = _sc_deg(dst_d)

    def unview(g):
        return g.reshape(N_NODES, HID)

    def view(a):
        return a.reshape(NC, NH, 2 * HID)

    xp = x.reshape(NH, 2 * IN_CH)

    g1 = _tc0(xp, W1, dp)
    acc1 = view(_sc_scatter(unview(g1), src, dst, zeros64))
    h1, g2 = _tc_mid(acc1, acc1, g1, dp, b1.reshape(1, HID), W2)
    acc2 = view(_sc_scatter(unview(g2), src, dst, zeros64))
    h2, g3 = _tc_mid(acc2, acc2, g2, dp, b2.reshape(1, HID), W3)
    acc3 = view(_sc_scatter(unview(g3), src, dst, zeros64))

    out = _tc_fin(acc3, acc3, g3, dp, b3.reshape(1, HID),
                  xp, h1, h2,
                  Wm1[:IN_CH], Wm1[IN_CH:IN_CH + HID],
                  Wm1[IN_CH + HID:IN_CH + 2 * HID], Wm1[IN_CH + 2 * HID:],
                  bm1.reshape(1, HID), Wm2, bm2.reshape(1, OUT_CH))
    return out.reshape(N_NODES, OUT_CH)


# speedup vs baseline: 2.8256x; 1.0184x over previous
import functools

import jax
import jax.numpy as jnp
from jax import lax
from jax.experimental import pallas as pl
from jax.experimental.pallas import tpu as pltpu
from jax.experimental.pallas import tpu_sc as plsc

N_NODES = 10000
N_EDGES = 320000
IN_CH = 128
HID = 64
OUT_CH = 64

NC, NS = 2, 16
NW = NC * NS
EPW = N_EDGES // NW
CHUNK = 125
NCHUNK = 80
NBUF = 4
ROW_BLK = 80
NROWBLK = N_NODES // ROW_BLK
RB = 2000
NBLK = N_NODES // RB
RBH = RB // 2
NH = N_NODES // 2
NHP = NH + 8

_MESH = plsc.VectorSubcoreMesh(core_axis_name="c", subcore_axis_name="s")


def _worker_ids():
    c = lax.axis_index("c")
    s = lax.axis_index("s")
    return c, s, c * NS + s


def _deg_body(dst_hbm, out_hbm, dstv, dege, dego, sem):
    c, s, w = _worker_ids()

    zeros = jnp.zeros((16,), jnp.float32)
    for k in range(NHP // 16):
        dege[pl.ds(k * 16, 16)] = zeros
        dego[pl.ds(k * 16, 16)] = zeros

    pltpu.sync_copy(dst_hbm.at[w], dstv)
    ones = jnp.ones((16,), jnp.float32)

    for g in range(EPW // 16):
        idx = dstv[pl.ds(g * 16, 16)]
        row = jax.lax.shift_right_logical(idx, 1)
        odd = jax.lax.eq(jax.lax.bitwise_and(idx, 1), 1)
        plsc.addupdate_scatter(dege, [row], ones, mask=jnp.logical_not(odd))
        plsc.addupdate_scatter(dego, [row], ones, mask=odd)

    for k in range(NBLK):
        pltpu.sync_copy(dege.at[pl.ds(k * RBH, RBH)], out_hbm.at[k, w, 0])
        pltpu.sync_copy(dego.at[pl.ds(k * RBH, RBH)], out_hbm.at[k, w, 1])


_sc_deg = pl.kernel(
    _deg_body,
    out_type=jax.ShapeDtypeStruct((NBLK, NW, 2, RBH), jnp.float32),
    mesh=_MESH,
    scratch_types=[
        pltpu.VMEM((EPW,), jnp.int32),
        pltpu.VMEM((NHP,), jnp.float32),
        pltpu.VMEM((NHP,), jnp.float32),
        pltpu.SemaphoreType.DMA,
    ],
    compiler_params=pltpu.CompilerParams(use_tc_tiling_on_sc=False,
                                         needs_layout_passes=False),
)


def _scatter_body(g_hbm, src_hbm, dst_hbm, zeros_hbm, out_hbm,
                  srcv, dstv, ra0, ra1, ra2, ra3, rb0, rb1, rb2, rb3,
                  zerosv, acc, gsa, gsb, ssa, ssb):
    c, s, w = _worker_ids()
    bufa = (ra0, ra1, ra2, ra3)
    bufb = (rb0, rb1, rb2, rb3)
    cps = pltpu.async_copy(src_hbm.at[w], srcv, gsa)
    cpd = pltpu.async_copy(dst_hbm.at[w], dstv, gsb)
    pltpu.sync_copy(zeros_hbm, zerosv)
    cps.wait()
    cpd.wait()

    for b in range(NBUF):
        pltpu.async_copy(g_hbm.at[srcv.at[b]], bufa[b], gsa)

    @pl.loop(s, NROWBLK, step=NS)
    def _zero(k):
        pltpu.async_copy(zerosv, acc.at[pl.ds(k * ROW_BLK, ROW_BLK)], ssa)

    @pl.loop(s, NROWBLK, step=NS)
    def _zerow(k):
        pltpu.make_async_copy(zerosv, acc.at[pl.ds(k * ROW_BLK, ROW_BLK)],
                              ssa).wait()

    plsc.subcore_barrier()

    @pl.loop(0, NCHUNK, step=2 * NBUF)
    def _edges(j):
        for b in range(NBUF):
            pltpu.async_copy(g_hbm.at[srcv.at[j + NBUF + b]], bufb[b], gsb)
        for b in range(NBUF):
            pltpu.make_async_copy(g_hbm.at[srcv.at[j + b]], bufa[b], gsa).wait()
        for b in range(NBUF):
            pltpu.async_copy(bufa[b], acc.at[dstv.at[j + b]], ssa, add=True)
        for b in range(NBUF):
            pltpu.make_async_copy(g_hbm.at[srcv.at[j + NBUF + b]], bufb[b], gsb).wait()
        for b in range(NBUF):
            pltpu.async_copy(bufb[b], acc.at[dstv.at[j + NBUF + b]], ssb, add=True)
        for b in range(NBUF):
            pltpu.make_async_copy(bufa[b], acc.at[dstv.at[j + b]], ssa).wait()

        @pl.when(j + 2 * NBUF < NCHUNK)
        def _prefetch_a():
            for b in range(NBUF):
                pltpu.async_copy(g_hbm.at[srcv.at[j + 2 * NBUF + b]], bufa[b], gsa)

        for b in range(NBUF):
            pltpu.make_async_copy(bufb[b], acc.at[dstv.at[j + NBUF + b]], ssb).wait()

    plsc.subcore_barrier()

    @pl.loop(s, NROWBLK, step=NS)
    def _out(k):
        pltpu.async_copy(acc.at[pl.ds(k * ROW_BLK, ROW_BLK)],
                         out_hbm.at[c, pl.ds(k * ROW_BLK, ROW_BLK)], ssa)

    @pl.loop(s, NROWBLK, step=NS)
    def _outw(k):
        pltpu.make_async_copy(acc.at[pl.ds(k * ROW_BLK, ROW_BLK)],
                              out_hbm.at[c, pl.ds(k * ROW_BLK, ROW_BLK)],
                              ssa).wait()


_sc_scatter = pl.kernel(
    _scatter_body,
    out_type=jax.ShapeDtypeStruct((NC, N_NODES, HID), jnp.float32),
    mesh=_MESH,
    scratch_types=(
        [pltpu.VMEM((NCHUNK, CHUNK), jnp.int32)] * 2
        + [pltpu.VMEM((CHUNK, HID), jnp.float32)] * (2 * NBUF)
        + [pltpu.VMEM((ROW_BLK, HID), jnp.float32),
           pltpu.VMEM_SHARED((N_NODES, HID), jnp.float32)]
        + [pltpu.SemaphoreType.DMA] * 4
    ),
    compiler_params=pltpu.CompilerParams(use_tc_tiling_on_sc=False),
)


GRID = (N_NODES // RB,)


def _dinv_pair(dp_ref):
    ones = jnp.ones((NW, 1), jnp.float32)
    dn = (((0,), (0,)), ((), ()))
    de = lax.dot_general(dp_ref[0, :, 0], ones, dn,
                         preferred_element_type=jnp.float32)
    do = lax.dot_general(dp_ref[0, :, 1], ones, dn,
                         preferred_element_type=jnp.float32)
    return lax.rsqrt(1.0 + de), lax.rsqrt(1.0 + do)


def _halves(ref):
    v = ref[...]
    n = v.shape[1] // 2
    return v[:, :n], v[:, n:]


def _tc0_body(x_ref, w_ref, dp_ref, g_ref):
    dve, dvo = _dinv_pair(dp_ref)
    xe, xo = _halves(x_ref)
    ge = dve * jnp.dot(xe, w_ref[...], preferred_element_type=jnp.float32)
    go = dvo * jnp.dot(xo, w_ref[...], preferred_element_type=jnp.float32)
    g_ref[...] = jnp.concatenate([ge, go], axis=1)


def _acc_halves(aa_ref, ab_ref):
    v = aa_ref[0] + ab_ref[0]
    return v[:, :HID], v[:, HID:]


def _tc_mid_body(aa_ref, ab_ref, g_ref, dp_ref, b_ref, w_ref,
                 h_ref, g2_ref):
    dve, dvo = _dinv_pair(dp_ref)
    ae, ao = _acc_halves(aa_ref, ab_ref)
    ge, go = _halves(g_ref)
    he = jnp.maximum(dve * (ae + ge) + b_ref[...], 0.0)
    ho = jnp.maximum(dvo * (ao + go) + b_ref[...], 0.0)
    h_ref[...] = jnp.concatenate([he, ho], axis=1)
    g2e = dve * jnp.dot(he, w_ref[...], preferred_element_type=jnp.float32)
    g2o = dvo * jnp.dot(ho, w_ref[...], preferred_element_type=jnp.float32)
    g2_ref[...] = jnp.concatenate([g2e, g2o], axis=1)


def _softmax(z):
    z -= jnp.max(z, axis=1, keepdims=True)
    ez = jnp.exp(z)
    return ez / jnp.sum(ez, axis=1, keepdims=True)


def _tc_fin_body(aa_ref, ab_ref, g_ref, dp_ref, b_ref,
                 x_ref, h1_ref, h2_ref, wx_ref, wh1_ref, wh2_ref, wh3_ref,
                 bm1_ref, wm2_ref, bm2_ref, out_ref):
    dve, dvo = _dinv_pair(dp_ref)
    ae, ao = _acc_halves(aa_ref, ab_ref)
    ge, go = _halves(g_ref)
    h3e = jnp.maximum(dve * (ae + ge) + b_ref[...], 0.0)
    h3o = jnp.maximum(dvo * (ao + go) + b_ref[...], 0.0)
    xe, xo = _halves(x_ref)
    h1e, h1o = _halves(h1_ref)
    h2e, h2o = _halves(h2_ref)
    f32 = jnp.float32
    me = (jnp.dot(xe, wx_ref[...], preferred_element_type=f32)
          + jnp.dot(h1e, wh1_ref[...], preferred_element_type=f32)
          + jnp.dot(h2e, wh2_ref[...], preferred_element_type=f32)
          + jnp.dot(h3e, wh3_ref[...], preferred_element_type=f32))
    mo = (jnp.dot(xo, wx_ref[...], preferred_element_type=f32)
          + jnp.dot(h1o, wh1_ref[...], preferred_element_type=f32)
          + jnp.dot(h2o, wh2_ref[...], preferred_element_type=f32)
          + jnp.dot(h3o, wh3_ref[...], preferred_element_type=f32))
    me = jnp.maximum(me + bm1_ref[...], 0.0)
    mo = jnp.maximum(mo + bm1_ref[...], 0.0)
    ze = jnp.dot(me, wm2_ref[...], preferred_element_type=f32) + bm2_ref[...]
    zo = jnp.dot(mo, wm2_ref[...], preferred_element_type=f32) + bm2_ref[...]
    out_ref[...] = jnp.concatenate([_softmax(ze), _softmax(zo)], axis=1)


def _rows(nc):
    return pl.BlockSpec((RB, nc), lambda i: (i, 0))


def _full(nr, nc):
    return pl.BlockSpec((nr, nc), lambda i: (0, 0))


def _degp():
    return pl.BlockSpec((1, NW, 2, RBH), lambda i: (i, 0, 0, 0))


def _acc_half(h):
    return pl.BlockSpec((1, RBH, 2 * HID), lambda i, _h=h: (_h, i, 0))


def _packed(nc):
    return pl.BlockSpec((RBH, 2 * nc), lambda i: (i, 0))


_tc0 = pl.pallas_call(
    _tc0_body,
    grid=GRID,
    in_specs=[_packed(IN_CH), _full(IN_CH, HID), _degp()],
    out_specs=_packed(HID),
    out_shape=jax.ShapeDtypeStruct((NH, 2 * HID), jnp.float32),
)

_tc_mid = pl.pallas_call(
    _tc_mid_body,
    grid=GRID,
    in_specs=[_acc_half(0), _acc_half(1), _packed(HID), _degp(),
              _full(1, HID), _full(HID, HID)],
    out_specs=[_packed(HID), _packed(HID)],
    out_shape=[jax.ShapeDtypeStruct((NH, 2 * HID), jnp.float32),
               jax.ShapeDtypeStruct((NH, 2 * HID), jnp.float32)],
)

_tc_fin = pl.pallas_call(
    _tc_fin_body,
    grid=GRID,
    in_specs=[_acc_half(0), _acc_half(1), _packed(HID), _degp(),
              _full(1, HID), _packed(IN_CH), _packed(HID), _packed(HID),
              _full(IN_CH, HID), _full(HID, HID), _full(HID, HID),
              _full(HID, HID), _full(1, HID), _full(HID, OUT_CH),
              _full(1, OUT_CH)],
    out_specs=_packed(OUT_CH),
    out_shape=jax.ShapeDtypeStruct((NH, 2 * OUT_CH), jnp.float32),
)


@jax.jit
def kernel(x, edge_index, W1, b1, W2, b2, W3, b3, Wm1, bm1, Wm2, bm2):
    src = edge_index[0].astype(jnp.int32).reshape(NW, NCHUNK, CHUNK)
    dst = edge_index[1].astype(jnp.int32).reshape(NW, NCHUNK, CHUNK)
    dst_d = edge_index[1].astype(jnp.int32).reshape(NW, EPW)

    zeros64 = jnp.zeros((ROW_BLK, HID), jnp.float32)

    dp = _sc_deg(dst_d)

    def unview(g):
        return g.reshape(N_NODES, HID)

    def view(a):
        return a.reshape(NC, NH, 2 * HID)

    xp = x.reshape(NH, 2 * IN_CH)

    g1 = _tc0(xp, W1, dp)
    acc1 = view(_sc_scatter(unview(g1), src, dst, zeros64))
    h1, g2 = _tc_mid(acc1, acc1, g1, dp, b1.reshape(1, HID), W2)
    acc2 = view(_sc_scatter(unview(g2), src, dst, zeros64))
    h2, g3 = _tc_mid(acc2, acc2, g2, dp, b2.reshape(1, HID), W3)
    acc3 = view(_sc_scatter(unview(g3), src, dst, zeros64))

    out = _tc_fin(acc3, acc3, g3, dp, b3.reshape(1, HID),
                  xp, h1, h2,
                  Wm1[:IN_CH], Wm1[IN_CH:IN_CH + HID],
                  Wm1[IN_CH + HID:IN_CH + 2 * HID], Wm1[IN_CH + 2 * HID:],
                  bm1.reshape(1, HID), Wm2, bm2.reshape(1, OUT_CH))
    return out.reshape(N_NODES, OUT_CH)
